# baseline (speedup 1.0000x reference)
import math

import jax
import jax.numpy as jnp
from jax import lax
from jax.experimental import pallas as pl
from jax.experimental.pallas import tpu as pltpu
from jax.experimental.pallas import tpu_sc as plsc

N = 10000
NP = 10240
E = 320000
EP = 327680
H = 128
NC = 2
NS = 16
CH = 64
SUP = 512
ECH = 128
ESUP = 2048
ROWS_C = NP // NS
ROWS_W = NP // (NC * NS)
EPT = EP // NC // NS
NR = NP // H


def _bcast16(ref, i):
    return plsc.load_gather(ref, [jnp.full((16,), i, jnp.int32)])


def _edge_body(src_hbm, dst_hbm, attr_hbm, m_hbm,
               vn_hbm, ew_hbm, degc_hbm, degs_hbm,
               m_v, srcb, dstb, attrb, vnb, ewb, dbuf, esem, degc_sh, degs_sh):
    c = lax.axis_index("c")
    s = lax.axis_index("s")

    pltpu.sync_copy(m_hbm, m_v)

    def _z(i, _):
        dbuf[pl.ds(i * 16, 16)] = jnp.zeros((16,), jnp.float32)
        return 0
    lax.fori_loop(0, ROWS_C // 16, _z, 0)
    sl = pl.ds(pl.multiple_of(s * ROWS_C, ROWS_C), ROWS_C)
    pltpu.sync_copy(dbuf, degc_sh.at[sl])
    pltpu.sync_copy(dbuf, degs_sh.at[sl])
    plsc.subcore_barrier()

    ebase = c * (EP // NC) + s * EPT

    def _super(sc, _):
        sbase = pl.multiple_of(ebase + sc * ESUP, ESUP)
        pltpu.sync_copy(src_hbm.at[pl.ds(sbase, ESUP)], srcb)
        pltpu.sync_copy(dst_hbm.at[pl.ds(pl.multiple_of(sbase // ECH, ESUP // ECH), ESUP // ECH)], dstb)
        pltpu.sync_copy(attr_hbm.at[pl.ds(sbase, ESUP)], attrb)

        def _chunk(ch, _):
            e0 = ch * ECH
            for g in range(ECH // 16):
                o = pl.ds(e0 + g * 16, 16)
                sidx = srcb[o]
                didx = dstb[ch, pl.ds(g * 16, 16)]
                ms = plsc.load_gather(m_v, [sidx])
                md = plsc.load_gather(m_v, [didx])
                vn16 = ms * md
                vnb[o] = vn16
                ewb[o] = attrb[o] * vn16
            pltpu.async_copy(ewb.at[pl.ds(e0, ECH)], degc_sh.at[dstb.at[ch]], esem.at[0], add=True)
            pltpu.async_copy(vnb.at[pl.ds(e0, ECH)], degs_sh.at[dstb.at[ch]], esem.at[1], add=True)
            return 0

        lax.fori_loop(0, ESUP // ECH, _chunk, 0)
        pltpu.sync_copy(vnb, vn_hbm.at[pl.ds(sbase, ESUP)])
        pltpu.sync_copy(ewb, ew_hbm.at[pl.ds(sbase, ESUP)])
        pltpu.make_async_copy(ew_hbm.at[pl.ds(sbase, ESUP)], ewb, esem.at[0]).wait()
        pltpu.make_async_copy(vn_hbm.at[pl.ds(sbase, ESUP)], vnb, esem.at[1]).wait()
        return 0

    lax.fori_loop(0, EPT // ESUP, _super, 0)
    plsc.subcore_barrier()
    pltpu.sync_copy(degc_sh.at[sl], degc_hbm.at[c, sl])
    pltpu.sync_copy(degs_sh.at[sl], degs_hbm.at[c, sl])


@jax.jit
def _edge_pass(src, dst2d, attr, m):
    mesh = plsc.VectorSubcoreMesh(core_axis_name="c", subcore_axis_name="s")
    return pl.kernel(
        _edge_body,
        out_type=(
            jax.ShapeDtypeStruct((EP,), jnp.float32),
            jax.ShapeDtypeStruct((EP,), jnp.float32),
            jax.ShapeDtypeStruct((NC, NP), jnp.float32),
            jax.ShapeDtypeStruct((NC, NP), jnp.float32),
        ),
        mesh=mesh,
        compiler_params=pltpu.CompilerParams(needs_layout_passes=False),
        scratch_types=[
            pltpu.VMEM((NP,), jnp.float32),
            pltpu.VMEM((ESUP,), jnp.int32),
            pltpu.VMEM((ESUP // ECH, ECH), jnp.int32),
            pltpu.VMEM((ESUP,), jnp.float32),
            pltpu.VMEM((ESUP,), jnp.float32),
            pltpu.VMEM((ESUP,), jnp.float32),
            pltpu.VMEM((ROWS_C,), jnp.float32),
            pltpu.SemaphoreType.DMA((2,)),
            pltpu.VMEM_SHARED((NP,), jnp.float32),
            pltpu.VMEM_SHARED((NP,), jnp.float32),
        ],
    )(src, dst2d, attr, m)


def _wide_body(src_hbm, dst_hbm, ew_hbm, xt_hbm, dinv_hbm, agg_hbm,
               srcb, dstb, ewb, rows3, dinv_v, scale_v, gsem, ssem, out_sh):
    c = lax.axis_index("c")
    s = lax.axis_index("s")

    pltpu.sync_copy(dinv_hbm, dinv_v)

    @plsc.parallel_loop(0, CH, unroll=4)
    def _zrow(e):
        for j in range(H // 16):
            rows3[0, e, pl.ds(j * 16, 16)] = jnp.zeros((16,), jnp.float32)

    def _zero_chunk(r, _):
        pltpu.sync_copy(rows3.at[0], out_sh.at[pl.ds(pl.multiple_of(s * ROWS_C + r * CH, CH), CH)])
        return 0

    lax.fori_loop(0, ROWS_C // CH, _zero_chunk, 0)

    plsc.subcore_barrier()

    ebase = c * (EP // NC) + s * EPT
    NCH = EPT // CH
    CPS = SUP // CH

    def _fire_gather(i):
        b = lax.rem(i, 4)
        sup = i // CPS
        sb = lax.rem(sup, 2)
        off = pl.multiple_of((i - sup * CPS) * CH, CH)
        idx = srcb.at[sb, pl.ds(off, CH)]
        pltpu.async_copy(xt_hbm.at[idx], rows3.at[b], gsem.at[b])

    def _wait_gather(i):
        b = lax.rem(i, 4)
        sup = i // CPS
        sb = lax.rem(sup, 2)
        off = pl.multiple_of((i - sup * CPS) * CH, CH)
        idx = srcb.at[sb, pl.ds(off, CH)]
        pltpu.make_async_copy(xt_hbm.at[idx], rows3.at[b], gsem.at[b]).wait()

    def _fire_scatter(i):
        b = lax.rem(i, 4)
        sup = i // CPS
        sb = lax.rem(sup, 2)
        loc = i - sup * CPS
        pltpu.async_copy(rows3.at[b], out_sh.at[dstb.at[sb, loc]], ssem.at[b], add=True)

    def _wait_scatter(i):
        b = lax.rem(i, 4)
        sup = i // CPS
        sb = lax.rem(sup, 2)
        loc = i - sup * CPS
        pltpu.make_async_copy(rows3.at[b], out_sh.at[dstb.at[sb, loc]], ssem.at[b]).wait()

    def _load_super(sc):
        sb = lax.rem(sc, 2)
        sbase = pl.multiple_of(ebase + sc * SUP, SUP)
        pltpu.sync_copy(src_hbm.at[pl.ds(sbase, SUP)], srcb.at[sb])
        pltpu.sync_copy(dst_hbm.at[pl.ds(pl.multiple_of(sbase // CH, CPS), CPS)], dstb.at[sb])
        pltpu.sync_copy(ew_hbm.at[pl.ds(sbase, SUP)], ewb.at[sb])

    _load_super(0)
    _fire_gather(0)

    def _chunk(i, _):
        b = lax.rem(i, 4)
        sup = i // CPS
        sb = lax.rem(sup, 2)
        loc = i - sup * CPS

        @pl.when((loc == CPS - 1) & (sup < EPT // SUP - 1))
        def _():
            _load_super(sup + 1)

        @pl.when(i >= 3)
        def _():
            _wait_scatter(i - 3)

        @pl.when(i < NCH - 1)
        def _():
            _fire_gather(i + 1)

        e0 = pl.multiple_of(loc * CH, CH)
        for g in range(CH // 16):
            o = pl.ds(e0 + g * 16, 16)
            sidx = srcb[sb, o]
            dsrc = plsc.load_gather(dinv_v, [sidx])
            scale_v[pl.ds(g * 16, 16)] = ewb[sb, o] * dsrc

        _wait_gather(i)

        @plsc.parallel_loop(0, CH, unroll=4)
        def _scale_row(e):
            bc = _bcast16(scale_v, e)
            for j in range(H // 16):
                rows3[b, e, pl.ds(j * 16, 16)] = rows3[b, e, pl.ds(j * 16, 16)] * bc

        _fire_scatter(i)
        return 0

    lax.fori_loop(0, NCH, _chunk, 0)
    _wait_scatter(NCH - 3)
    _wait_scatter(NCH - 2)
    _wait_scatter(NCH - 1)

    plsc.subcore_barrier()

    sl = pl.ds(pl.multiple_of(s * ROWS_C, ROWS_C), ROWS_C)
    pltpu.sync_copy(out_sh.at[sl], agg_hbm.at[c, sl])


@jax.jit
def _wide_pass(src, dst2d, ew, xt, dinv):
    mesh = plsc.VectorSubcoreMesh(core_axis_name="c", subcore_axis_name="s")
    return pl.kernel(
        _wide_body,
        out_type=jax.ShapeDtypeStruct((NC, NP, H), jnp.float32),
        mesh=mesh,
        compiler_params=pltpu.CompilerParams(needs_layout_passes=False),
        scratch_types=[
            pltpu.VMEM((2, SUP), jnp.int32),
            pltpu.VMEM((2, SUP // CH, CH), jnp.int32),
            pltpu.VMEM((2, SUP), jnp.float32),
            pltpu.VMEM((4, CH, H), jnp.float32),
            pltpu.VMEM((NP,), jnp.float32),
            pltpu.VMEM((CH,), jnp.float32),
            pltpu.SemaphoreType.DMA((4,)),
            pltpu.SemaphoreType.DMA((4,)),
            pltpu.VMEM_SHARED((NP, H), jnp.float32),
        ],
    )(src, dst2d, ew, xt, dinv)


def _score_body(src_hbm, dst_hbm, vn_hbm, st_hbm, dinv_hbm, sagg_hbm,
                st_v, dinv_v, srcb, dstb, vnb, valb, dbuf, esem, sagg_sh):
    c = lax.axis_index("c")
    s = lax.axis_index("s")

    pltpu.sync_copy(st_hbm, st_v)
    pltpu.sync_copy(dinv_hbm, dinv_v)

    def _z(i, _):
        dbuf[pl.ds(i * 16, 16)] = jnp.zeros((16,), jnp.float32)
        return 0
    lax.fori_loop(0, ROWS_C // 16, _z, 0)
    sl = pl.ds(pl.multiple_of(s * ROWS_C, ROWS_C), ROWS_C)
    pltpu.sync_copy(dbuf, sagg_sh.at[sl])
    plsc.subcore_barrier()

    ebase = c * (EP // NC) + s * EPT

    def _super(sc, _):
        sbase = pl.multiple_of(ebase + sc * ESUP, ESUP)
        pltpu.sync_copy(src_hbm.at[pl.ds(sbase, ESUP)], srcb)
        pltpu.sync_copy(dst_hbm.at[pl.ds(pl.multiple_of(sbase // ECH, ESUP // ECH), ESUP // ECH)], dstb)
        pltpu.sync_copy(vn_hbm.at[pl.ds(sbase, ESUP)], vnb)

        def _chunk(ch, _):
            e0 = ch * ECH
            for g in range(ECH // 16):
                o = pl.ds(e0 + g * 16, 16)
                sidx = srcb[o]
                stg = plsc.load_gather(st_v, [sidx])
                dg = plsc.load_gather(dinv_v, [sidx])
                valb[o] = vnb[o] * stg * dg
            pltpu.async_copy(valb.at[pl.ds(e0, ECH)], sagg_sh.at[dstb.at[ch]], esem, add=True)
            return 0

        lax.fori_loop(0, ESUP // ECH, _chunk, 0)
        pltpu.make_async_copy(src_hbm.at[pl.ds(sbase, ESUP)], valb, esem).wait()
        return 0

    lax.fori_loop(0, EPT // ESUP, _super, 0)
    plsc.subcore_barrier()
    pltpu.sync_copy(sagg_sh.at[sl], sagg_hbm.at[c, sl])


@jax.jit
def _score_pass(src, dst2d, vn, st, dinvs):
    mesh = plsc.VectorSubcoreMesh(core_axis_name="c", subcore_axis_name="s")
    return pl.kernel(
        _score_body,
        out_type=jax.ShapeDtypeStruct((NC, NP), jnp.float32),
        mesh=mesh,
        compiler_params=pltpu.CompilerParams(needs_layout_passes=False),
        scratch_types=[
            pltpu.VMEM((NP,), jnp.float32),
            pltpu.VMEM((NP,), jnp.float32),
            pltpu.VMEM((ESUP,), jnp.int32),
            pltpu.VMEM((ESUP // ECH, ECH), jnp.int32),
            pltpu.VMEM((ESUP,), jnp.float32),
            pltpu.VMEM((ESUP,), jnp.float32),
            pltpu.VMEM((ROWS_C,), jnp.float32),
            pltpu.SemaphoreType.DMA,
            pltpu.VMEM_SHARED((NP,), jnp.float32),
        ],
    )(src, dst2d, vn, st, dinvs)


_PCH = 64


def _pool_body(hn_hbm, g_hbm, sel_hbm, hp_hbm, pmax_hbm, psum_hbm,
               rows_v, gbuf, selbuf, obuf):
    c = lax.axis_index("c")
    s = lax.axis_index("s")
    wid = c * NS + s
    base = wid * ROWS_W

    neg = jnp.full((16,), -3.0e38, jnp.float32)
    zero = jnp.zeros((16,), jnp.float32)
    acc0 = (neg,) * (H // 16) + (zero,) * (H // 16)

    def _chunk(r, acc):
        row0 = pl.multiple_of(base + r * _PCH, _PCH)
        pltpu.sync_copy(hn_hbm.at[pl.ds(row0, _PCH)], rows_v)
        pltpu.sync_copy(g_hbm.at[pl.ds(row0, _PCH)], gbuf)
        pltpu.sync_copy(sel_hbm.at[pl.ds(row0, _PCH)], selbuf)

        def _row(e, acc):
            bc = _bcast16(gbuf, e)
            selm = _bcast16(selbuf, e) > 0.0
            new = []
            for j in range(H // 16):
                v = rows_v[e, pl.ds(j * 16, 16)] * bc
                rows_v[e, pl.ds(j * 16, 16)] = v
                mx = acc[j]
                new.append(jnp.where(selm, jnp.maximum(mx, v), mx))
            for j in range(H // 16):
                v = rows_v[e, pl.ds(j * 16, 16)]
                new.append(acc[H // 16 + j] + v)
            return tuple(new)

        acc = lax.fori_loop(0, _PCH, _row, acc)
        pltpu.sync_copy(rows_v, hp_hbm.at[pl.ds(row0, _PCH)])
        return acc

    acc = lax.fori_loop(0, ROWS_W // _PCH, _chunk, acc0)

    for j in range(H // 16):
        obuf[pl.ds(j * 16, 16)] = acc[j]
    pltpu.sync_copy(obuf, pmax_hbm.at[c, s])
    for j in range(H // 16):
        obuf[pl.ds(j * 16, 16)] = acc[H // 16 + j]
    pltpu.sync_copy(obuf, psum_hbm.at[c, s])


@jax.jit
def _pool_pass(hn, g, sel):
    mesh = plsc.VectorSubcoreMesh(core_axis_name="c", subcore_axis_name="s")
    return pl.kernel(
        _pool_body,
        out_type=(
            jax.ShapeDtypeStruct((NP, H), jnp.float32),
            jax.ShapeDtypeStruct((NC, NS, H), jnp.float32),
            jax.ShapeDtypeStruct((NC, NS, H), jnp.float32),
        ),
        mesh=mesh,
        compiler_params=pltpu.CompilerParams(needs_layout_passes=False),
        scratch_types=[
            pltpu.VMEM((_PCH, H), jnp.float32),
            pltpu.VMEM((_PCH,), jnp.float32),
            pltpu.VMEM((_PCH,), jnp.float32),
            pltpu.VMEM((H,), jnp.float32),
        ],
    )(hn, g, sel)


def _mm1_body(h_ref, w_ref, degc_ref, degs_ref, xt_ref, dc_ref, ds_ref):
    xt_ref[...] = jnp.dot(h_ref[...], w_ref[...], preferred_element_type=jnp.float32)
    dc_ref[...] = lax.rsqrt(degc_ref[0] + degc_ref[1] + 1.0)
    ds_ref[...] = lax.rsqrt(degs_ref[0] + degs_ref[1] + 1.0)


@jax.jit
def _mm1_prep(h, W, degc2, degs2):
    return pl.pallas_call(
        _mm1_body,
        out_shape=(
            jax.ShapeDtypeStruct((NP, H), jnp.float32),
            jax.ShapeDtypeStruct((NR, H), jnp.float32),
            jax.ShapeDtypeStruct((NR, H), jnp.float32),
        ),
    )(h, W, degc2, degs2)


def _mm2_body(agg_ref, xt_ref, dc_ref, b_ref, wp_ref, hn_ref, st_ref):
    dc = dc_ref[...]
    hn = jax.nn.relu(dc * (agg_ref[0] + agg_ref[1]) + dc * dc * xt_ref[...]
                     + b_ref[...][None, :])
    hn_ref[...] = hn
    st_ref[...] = jnp.dot(hn, wp_ref[...], preferred_element_type=jnp.float32)


@jax.jit
def _mm2(agg, xt, dc_col, b, Wp):
    return pl.pallas_call(
        _mm2_body,
        out_shape=(
            jax.ShapeDtypeStruct((NP, H), jnp.float32),
            jax.ShapeDtypeStruct((NP, 1), jnp.float32),
        ),
    )(agg, xt, dc_col, b, Wp)


def _topk_body(k, sagg_ref, st_ref, dinvs_ref, m_ref, bp_ref, sel_ref, g_ref):
    dinvs = dinvs_ref[...]
    st = st_ref[...]
    score = dinvs * (sagg_ref[0] + sagg_ref[1]) + dinvs * dinvs * st + bp_ref[0, 0]
    scorem = jnp.where(m_ref[...] > 0, score, -jnp.inf)

    bits = lax.bitcast_convert_type(scorem, jnp.int32)
    key = jnp.where(bits >= 0, bits, bits ^ jnp.int32(0x7FFFFFFF))

    cnt_nonneg = jnp.sum((key >= 0).astype(jnp.int32))
    t0 = jnp.where(cnt_nonneg >= k, jnp.int32(0), jnp.int32(-2147483648))

    def _tstep(i, t):
        cand = t | lax.shift_left(jnp.int32(1), jnp.int32(30) - i)
        cnt = jnp.sum((key >= cand).astype(jnp.int32))
        return jnp.where(cnt >= k, cand, t)

    t = lax.fori_loop(0, 31, _tstep, t0)

    cnt_gt = jnp.sum((key > t).astype(jnp.int32))
    need = k - cnt_gt
    idx = (lax.broadcasted_iota(jnp.int32, (NR, H), 0) * H
           + lax.broadcasted_iota(jnp.int32, (NR, H), 1))
    eq = key == t

    def _jstep(i, jv):
        cand = jv | lax.shift_left(jnp.int32(1), jnp.int32(13) - i)
        cnt = jnp.sum((eq & (idx < cand)).astype(jnp.int32))
        return jnp.where(cnt <= need, cand, jv)

    j0 = lax.fori_loop(0, 14, _jstep, jnp.int32(0))

    sel = (key > t) | (eq & (idx < j0))
    self32 = sel.astype(jnp.float32)
    sel_ref[...] = self32
    g_ref[...] = jnp.tanh(score) * self32


import functools


@functools.partial(jax.jit, static_argnums=0)
def _topk_g(k, sagg2, st2, dinvs2, m2, bp):
    return pl.pallas_call(
        functools.partial(_topk_body, k),
        out_shape=(
            jax.ShapeDtypeStruct((NR, H), jnp.float32),
            jax.ShapeDtypeStruct((NR, H), jnp.float32),
        ),
    )(sagg2, st2, dinvs2, m2, bp)


def _head_body(k1, k2, k3, pmax_ref, psum_ref, wl1_ref, bl1_ref, wl2_ref,
               bl2_ref, wl3_ref, bl3_ref, o_ref):
    zs = []
    for l, kk in enumerate((k1, k2, k3)):
        mx = jnp.max(pmax_ref[l], axis=0, keepdims=True)
        mean = jnp.sum(psum_ref[l], axis=0, keepdims=True) / kk
        zs.append(jnp.concatenate([mx, mean], axis=1))
    z = zs[0] + zs[1] + zs[2]
    z = jax.nn.relu(z @ wl1_ref[...] + bl1_ref[...][None, :])
    z = jax.nn.relu(z @ wl2_ref[...] + bl2_ref[...][None, :])
    z = z @ wl3_ref[...] + bl3_ref[...][None, :]
    o_ref[...] = jax.nn.log_softmax(z, axis=-1)


@functools.partial(jax.jit, static_argnums=(0, 1, 2))
def _head(k1, k2, k3, pmax3, psum3, Wl1, bl1, Wl2, bl2, Wl3, bl3):
    return pl.pallas_call(
        functools.partial(_head_body, k1, k2, k3),
        out_shape=jax.ShapeDtypeStruct((1, 2), jnp.float32),
    )(pmax3, psum3, Wl1, bl1, Wl2, bl2, Wl3, bl3)


def _pad_nodes(a):
    return jnp.pad(a, ((0, NP - N),) + ((0, 0),) * (a.ndim - 1))


def kernel(x, edge_index, edge_attr, batch, W1, b1, Wp1, bp1, W2, b2, Wp2, bp2, W3, b3, Wp3, bp3, Wl1, bl1, Wl2, bl2, Wl3, bl3):
    src = edge_index[0].astype(jnp.int32)
    dst = edge_index[1].astype(jnp.int32)
    pad_n = EP - E
    pad_idx = N + (jnp.arange(pad_n, dtype=jnp.int32) % (NP - N))
    src_p = jnp.concatenate([src, pad_idx])
    dst_p = jnp.concatenate([dst, pad_idx])
    dst2dw = dst_p.reshape(EP // CH, CH)
    dst2de = dst_p.reshape(EP // ECH, ECH)
    attr_p = jnp.concatenate([edge_attr, jnp.zeros((pad_n,), jnp.float32)])
    ones_p = jnp.concatenate([jnp.ones((E,), jnp.float32), jnp.zeros((pad_n,), jnp.float32)])

    m = _pad_nodes(jnp.ones((N,), jnp.float32))
    h = _pad_nodes(x)

    ks, n = [], N
    for _ in range(3):
        ks.append(int(math.ceil(0.8 * n)))
        n = ks[-1]
    Ws = [(W1, b1, Wp1, bp1, attr_p), (W2, b2, Wp2, bp2, attr_p),
          (W3, b3, Wp3, bp3, ones_p)]

    pmaxs, psums = [], []
    for li, (W, b, Wp, bp, attr_l) in enumerate(Ws):
        k = ks[li]
        vn, ew, degc2, degs2 = _edge_pass(src_p, dst2de, attr_l, m)
        xt, dc2, ds2 = _mm1_prep(h, W, degc2.reshape(NC, NR, H), degs2.reshape(NC, NR, H))
        dinv_c = dc2.reshape(NP)
        dinv_s = ds2.reshape(NP)
        agg = _wide_pass(src_p, dst2dw, ew, xt, dinv_c)
        hn, st = _mm2(agg, xt, jnp.broadcast_to(dinv_c.reshape(NP, 1), (NP, H)), b, Wp)
        st_flat = st.reshape(NP)
        sagg2 = _score_pass(src_p, dst2de, vn, st_flat, dinv_s)
        sel2, g2 = _topk_g(k, sagg2.reshape(NC, NR, H), st_flat.reshape(NR, H),
                           ds2, m.reshape(NR, H), bp.reshape(1, 1))
        sel = sel2.reshape(NP)
        h, pmax, psum = _pool_pass(hn, g2.reshape(NP), sel)
        pmaxs.append(pmax.reshape(NC * NS, H))
        psums.append(psum.reshape(NC * NS, H))
        m = sel

    return _head(float(ks[0]), float(ks[1]), float(ks[2]),
                 jnp.stack(pmaxs), jnp.stack(psums), Wl1, bl1, Wl2, bl2, Wl3, bl3)


# speedup vs baseline: 1.1352x; 1.1352x over previous
import math

import jax
import jax.numpy as jnp
from jax import lax
from jax.experimental import pallas as pl
from jax.experimental.pallas import tpu as pltpu
from jax.experimental.pallas import tpu_sc as plsc

N = 10000
NP = 10240
E = 320000
EP = 327680
H = 128
NC = 2
NS = 16
CH = 64
SUP = 512
ECH = 128
ESUP = 2048
ROWS_C = NP // NS
ROWS_W = NP // (NC * NS)
EPT = EP // NC // NS
NR = NP // H


def _bcast16(ref, i):
    return plsc.load_gather(ref, [jnp.full((16,), i, jnp.int32)])


def _edge_body(src_hbm, dst_hbm, attr_hbm, m_hbm,
               vn_hbm, ew_hbm, degc_hbm, degs_hbm, csrc_hbm, cdst_hbm,
               cew_hbm, cnt_hbm,
               m_v, srcb, dstb, attrb, vnb, ewb, dbuf, csrcb, cdstb, cewb,
               cntb, esem, degc_sh, degs_sh):
    c = lax.axis_index("c")
    s = lax.axis_index("s")

    pltpu.sync_copy(m_hbm, m_v)

    trash = N + lax.rem(lax.iota(jnp.int32, 16) * 17 + s * 16, NP - N)
    zero16 = jnp.zeros((16,), jnp.float32)

    def _tfill(i, _):
        o = pl.ds(i * 16, 16)
        csrcb[o] = trash
        cdstb[o] = trash
        cewb[o] = zero16
        return 0
    lax.fori_loop(0, (EPT + 64) // 16, _tfill, 0)

    def _z(i, _):
        dbuf[pl.ds(i * 16, 16)] = jnp.zeros((16,), jnp.float32)
        return 0
    lax.fori_loop(0, ROWS_C // 16, _z, 0)
    sl = pl.ds(pl.multiple_of(s * ROWS_C, ROWS_C), ROWS_C)
    pltpu.sync_copy(dbuf, degc_sh.at[sl])
    pltpu.sync_copy(dbuf, degs_sh.at[sl])
    plsc.subcore_barrier()

    ebase = c * (EP // NC) + s * EPT

    def _super(sc, coff):
        sbase = pl.multiple_of(ebase + sc * ESUP, ESUP)
        pltpu.sync_copy(src_hbm.at[pl.ds(sbase, ESUP)], srcb)
        pltpu.sync_copy(dst_hbm.at[pl.ds(pl.multiple_of(sbase // ECH, ESUP // ECH), ESUP // ECH)], dstb)
        pltpu.sync_copy(attr_hbm.at[pl.ds(sbase, ESUP)], attrb)

        def _chunk(ch, coff):
            e0 = ch * ECH
            for g in range(ECH // 16):
                o = pl.ds(e0 + g * 16, 16)
                sidx = srcb[o]
                didx = dstb[ch, pl.ds(g * 16, 16)]
                ms = plsc.load_gather(m_v, [sidx])
                md = plsc.load_gather(m_v, [didx])
                vn16 = ms * md
                ew16 = attrb[o] * vn16
                vnb[o] = vn16
                ewb[o] = ew16
                msk = ew16 != 0.0
                ow = pl.ds(coff, 16)
                plsc.store_compressed(csrcb.at[ow], sidx, mask=msk)
                plsc.store_compressed(cdstb.at[ow], didx, mask=msk)
                plsc.store_compressed(cewb.at[ow], ew16, mask=msk)
                coff = coff + jnp.sum(msk.astype(jnp.int32))
            pltpu.async_copy(ewb.at[pl.ds(e0, ECH)], degc_sh.at[dstb.at[ch]], esem.at[0], add=True)
            pltpu.async_copy(vnb.at[pl.ds(e0, ECH)], degs_sh.at[dstb.at[ch]], esem.at[1], add=True)
            return coff

        coff = lax.fori_loop(0, ESUP // ECH, _chunk, coff)
        pltpu.sync_copy(vnb, vn_hbm.at[pl.ds(sbase, ESUP)])
        pltpu.sync_copy(ewb, ew_hbm.at[pl.ds(sbase, ESUP)])
        pltpu.make_async_copy(ew_hbm.at[pl.ds(sbase, ESUP)], ewb, esem.at[0]).wait()
        pltpu.make_async_copy(vn_hbm.at[pl.ds(sbase, ESUP)], vnb, esem.at[1]).wait()
        return coff

    coff = lax.fori_loop(0, EPT // ESUP, _super, 0)
    cpad = ((coff + CH - 1) // CH) * CH
    cntb[...] = jnp.full((16,), cpad, jnp.int32)
    tbase = pl.multiple_of((c * NS + s) * EPT, EPT)
    pltpu.sync_copy(csrcb.at[pl.ds(0, EPT)], csrc_hbm.at[pl.ds(tbase, EPT)])
    pltpu.sync_copy(cdstb.at[pl.ds(0, EPT)], cdst_hbm.at[pl.ds(tbase, EPT)])
    pltpu.sync_copy(cewb.at[pl.ds(0, EPT)], cew_hbm.at[pl.ds(tbase, EPT)])
    pltpu.sync_copy(cntb, cnt_hbm.at[c * NS + s])
    plsc.subcore_barrier()
    pltpu.sync_copy(degc_sh.at[sl], degc_hbm.at[c, sl])
    pltpu.sync_copy(degs_sh.at[sl], degs_hbm.at[c, sl])


@jax.jit
def _edge_pass(src, dst2d, attr, m):
    mesh = plsc.VectorSubcoreMesh(core_axis_name="c", subcore_axis_name="s")
    return pl.kernel(
        _edge_body,
        out_type=(
            jax.ShapeDtypeStruct((EP,), jnp.float32),
            jax.ShapeDtypeStruct((EP,), jnp.float32),
            jax.ShapeDtypeStruct((NC, NP), jnp.float32),
            jax.ShapeDtypeStruct((NC, NP), jnp.float32),
            jax.ShapeDtypeStruct((EP,), jnp.int32),
            jax.ShapeDtypeStruct((EP,), jnp.int32),
            jax.ShapeDtypeStruct((EP,), jnp.float32),
            jax.ShapeDtypeStruct((NC * NS, 16), jnp.int32),
        ),
        mesh=mesh,
        compiler_params=pltpu.CompilerParams(needs_layout_passes=False),
        scratch_types=[
            pltpu.VMEM((NP,), jnp.float32),
            pltpu.VMEM((ESUP,), jnp.int32),
            pltpu.VMEM((ESUP // ECH, ECH), jnp.int32),
            pltpu.VMEM((ESUP,), jnp.float32),
            pltpu.VMEM((ESUP,), jnp.float32),
            pltpu.VMEM((ESUP,), jnp.float32),
            pltpu.VMEM((ROWS_C,), jnp.float32),
            pltpu.VMEM((EPT + 64,), jnp.int32),
            pltpu.VMEM((EPT + 64,), jnp.int32),
            pltpu.VMEM((EPT + 64,), jnp.float32),
            pltpu.VMEM((16,), jnp.int32),
            pltpu.SemaphoreType.DMA((2,)),
            pltpu.VMEM_SHARED((NP,), jnp.float32),
            pltpu.VMEM_SHARED((NP,), jnp.float32),
        ],
    )(src, dst2d, attr, m)


def _wide_body(src_hbm, dst_hbm, ew_hbm, xt_hbm, dinv_hbm, cnt_hbm, agg_hbm,
               srcb, dstb, ewb, rows3, dinv_v, scale_v, cnt_smem, gsem, ssem, out_sh):
    c = lax.axis_index("c")
    s = lax.axis_index("s")

    pltpu.sync_copy(dinv_hbm, dinv_v)

    @plsc.parallel_loop(0, CH, unroll=4)
    def _zrow(e):
        for j in range(H // 16):
            rows3[0, e, pl.ds(j * 16, 16)] = jnp.zeros((16,), jnp.float32)

    def _zero_chunk(r, _):
        pltpu.sync_copy(rows3.at[0], out_sh.at[pl.ds(pl.multiple_of(s * ROWS_C + r * CH, CH), CH)])
        return 0

    lax.fori_loop(0, ROWS_C // CH, _zero_chunk, 0)

    plsc.subcore_barrier()

    ebase = pl.multiple_of((c * NS + s) * EPT, EPT)
    pltpu.sync_copy(cnt_hbm.at[c * NS + s], cnt_smem)
    nch = jnp.max(cnt_smem[...]) // CH
    CPS = SUP // CH

    def _fire_gather(i):
        b = lax.rem(i, 4)
        sup = i // CPS
        sb = lax.rem(sup, 2)
        off = pl.multiple_of((i - sup * CPS) * CH, CH)
        idx = srcb.at[sb, pl.ds(off, CH)]
        pltpu.async_copy(xt_hbm.at[idx], rows3.at[b], gsem.at[b])

    def _wait_gather(i):
        b = lax.rem(i, 4)
        sup = i // CPS
        sb = lax.rem(sup, 2)
        off = pl.multiple_of((i - sup * CPS) * CH, CH)
        idx = srcb.at[sb, pl.ds(off, CH)]
        pltpu.make_async_copy(xt_hbm.at[idx], rows3.at[b], gsem.at[b]).wait()

    def _fire_scatter(i):
        b = lax.rem(i, 4)
        sup = i // CPS
        sb = lax.rem(sup, 2)
        loc = i - sup * CPS
        pltpu.async_copy(rows3.at[b], out_sh.at[dstb.at[sb, loc]], ssem.at[b], add=True)

    def _wait_scatter(i):
        b = lax.rem(i, 4)
        sup = i // CPS
        sb = lax.rem(sup, 2)
        loc = i - sup * CPS
        pltpu.make_async_copy(rows3.at[b], out_sh.at[dstb.at[sb, loc]], ssem.at[b]).wait()

    def _load_super(sc):
        sb = lax.rem(sc, 2)
        sbase = pl.multiple_of(ebase + sc * SUP, SUP)
        pltpu.sync_copy(src_hbm.at[pl.ds(sbase, SUP)], srcb.at[sb])
        pltpu.sync_copy(dst_hbm.at[pl.ds(pl.multiple_of(sbase // CH, CPS), CPS)], dstb.at[sb])
        pltpu.sync_copy(ew_hbm.at[pl.ds(sbase, SUP)], ewb.at[sb])

    _load_super(0)

    @pl.when(nch > 0)
    def _():
        _fire_gather(0)

    def _chunk(i, _):
        b = lax.rem(i, 4)
        sup = i // CPS
        sb = lax.rem(sup, 2)
        loc = i - sup * CPS

        @pl.when((loc == CPS - 1) & (sup < EPT // SUP - 1))
        def _():
            _load_super(sup + 1)

        @pl.when(i >= 3)
        def _():
            _wait_scatter(i - 3)

        @pl.when(i < nch - 1)
        def _():
            _fire_gather(i + 1)

        e0 = pl.multiple_of(loc * CH, CH)
        for g in range(CH // 16):
            o = pl.ds(e0 + g * 16, 16)
            sidx = srcb[sb, o]
            dsrc = plsc.load_gather(dinv_v, [sidx])
            scale_v[pl.ds(g * 16, 16)] = ewb[sb, o] * dsrc

        _wait_gather(i)

        @plsc.parallel_loop(0, CH, unroll=4)
        def _scale_row(e):
            bc = _bcast16(scale_v, e)
            for j in range(H // 16):
                rows3[b, e, pl.ds(j * 16, 16)] = rows3[b, e, pl.ds(j * 16, 16)] * bc

        _fire_scatter(i)
        return 0

    lax.fori_loop(0, nch, _chunk, 0)
    for dd in (3, 2, 1):
        @pl.when(nch >= dd)
        def _(dd=dd):
            _wait_scatter(nch - dd)

    plsc.subcore_barrier()

    sl = pl.ds(pl.multiple_of(s * ROWS_C, ROWS_C), ROWS_C)
    pltpu.sync_copy(out_sh.at[sl], agg_hbm.at[c, sl])


@jax.jit
def _wide_pass(src, dst2d, ew, xt, dinv, cnts):
    mesh = plsc.VectorSubcoreMesh(core_axis_name="c", subcore_axis_name="s")
    return pl.kernel(
        _wide_body,
        out_type=jax.ShapeDtypeStruct((NC, NP, H), jnp.float32),
        mesh=mesh,
        compiler_params=pltpu.CompilerParams(needs_layout_passes=False),
        scratch_types=[
            pltpu.VMEM((2, SUP), jnp.int32),
            pltpu.VMEM((2, SUP // CH, CH), jnp.int32),
            pltpu.VMEM((2, SUP), jnp.float32),
            pltpu.VMEM((4, CH, H), jnp.float32),
            pltpu.VMEM((NP,), jnp.float32),
            pltpu.VMEM((CH,), jnp.float32),
            pltpu.VMEM((16,), jnp.int32),
            pltpu.SemaphoreType.DMA((4,)),
            pltpu.SemaphoreType.DMA((4,)),
            pltpu.VMEM_SHARED((NP, H), jnp.float32),
        ],
    )(src, dst2d, ew, xt, dinv, cnts)


def _score_body(src_hbm, dst_hbm, vn_hbm, st_hbm, dinv_hbm, sagg_hbm,
                st_v, dinv_v, srcb, dstb, vnb, valb, dbuf, esem, sagg_sh):
    c = lax.axis_index("c")
    s = lax.axis_index("s")

    pltpu.sync_copy(st_hbm, st_v)
    pltpu.sync_copy(dinv_hbm, dinv_v)

    def _z(i, _):
        dbuf[pl.ds(i * 16, 16)] = jnp.zeros((16,), jnp.float32)
        return 0
    lax.fori_loop(0, ROWS_C // 16, _z, 0)
    sl = pl.ds(pl.multiple_of(s * ROWS_C, ROWS_C), ROWS_C)
    pltpu.sync_copy(dbuf, sagg_sh.at[sl])
    plsc.subcore_barrier()

    ebase = c * (EP // NC) + s * EPT

    def _super(sc, _):
        sbase = pl.multiple_of(ebase + sc * ESUP, ESUP)
        pltpu.sync_copy(src_hbm.at[pl.ds(sbase, ESUP)], srcb)
        pltpu.sync_copy(dst_hbm.at[pl.ds(pl.multiple_of(sbase // ECH, ESUP // ECH), ESUP // ECH)], dstb)
        pltpu.sync_copy(vn_hbm.at[pl.ds(sbase, ESUP)], vnb)

        def _chunk(ch, _):
            e0 = ch * ECH
            for g in range(ECH // 16):
                o = pl.ds(e0 + g * 16, 16)
                sidx = srcb[o]
                stg = plsc.load_gather(st_v, [sidx])
                dg = plsc.load_gather(dinv_v, [sidx])
                valb[o] = vnb[o] * stg * dg
            pltpu.async_copy(valb.at[pl.ds(e0, ECH)], sagg_sh.at[dstb.at[ch]], esem, add=True)
            return 0

        lax.fori_loop(0, ESUP // ECH, _chunk, 0)
        pltpu.make_async_copy(src_hbm.at[pl.ds(sbase, ESUP)], valb, esem).wait()
        return 0

    lax.fori_loop(0, EPT // ESUP, _super, 0)
    plsc.subcore_barrier()
    pltpu.sync_copy(sagg_sh.at[sl], sagg_hbm.at[c, sl])


@jax.jit
def _score_pass(src, dst2d, vn, st, dinvs):
    mesh = plsc.VectorSubcoreMesh(core_axis_name="c", subcore_axis_name="s")
    return pl.kernel(
        _score_body,
        out_type=jax.ShapeDtypeStruct((NC, NP), jnp.float32),
        mesh=mesh,
        compiler_params=pltpu.CompilerParams(needs_layout_passes=False),
        scratch_types=[
            pltpu.VMEM((NP,), jnp.float32),
            pltpu.VMEM((NP,), jnp.float32),
            pltpu.VMEM((ESUP,), jnp.int32),
            pltpu.VMEM((ESUP // ECH, ECH), jnp.int32),
            pltpu.VMEM((ESUP,), jnp.float32),
            pltpu.VMEM((ESUP,), jnp.float32),
            pltpu.VMEM((ROWS_C,), jnp.float32),
            pltpu.SemaphoreType.DMA,
            pltpu.VMEM_SHARED((NP,), jnp.float32),
        ],
    )(src, dst2d, vn, st, dinvs)


_PCH = 64


def _pool_body(hn_hbm, g_hbm, sel_hbm, hp_hbm, pmax_hbm, psum_hbm,
               rows_v, gbuf, selbuf, obuf):
    c = lax.axis_index("c")
    s = lax.axis_index("s")
    wid = c * NS + s
    base = wid * ROWS_W

    neg = jnp.full((16,), -3.0e38, jnp.float32)
    zero = jnp.zeros((16,), jnp.float32)
    acc0 = (neg,) * (H // 16) + (zero,) * (H // 16)

    def _chunk(r, acc):
        row0 = pl.multiple_of(base + r * _PCH, _PCH)
        pltpu.sync_copy(hn_hbm.at[pl.ds(row0, _PCH)], rows_v)
        pltpu.sync_copy(g_hbm.at[pl.ds(row0, _PCH)], gbuf)
        pltpu.sync_copy(sel_hbm.at[pl.ds(row0, _PCH)], selbuf)

        def _row(e, acc):
            bc = _bcast16(gbuf, e)
            selm = _bcast16(selbuf, e) > 0.0
            new = []
            for j in range(H // 16):
                v = rows_v[e, pl.ds(j * 16, 16)] * bc
                rows_v[e, pl.ds(j * 16, 16)] = v
                mx = acc[j]
                new.append(jnp.where(selm, jnp.maximum(mx, v), mx))
            for j in range(H // 16):
                v = rows_v[e, pl.ds(j * 16, 16)]
                new.append(acc[H // 16 + j] + v)
            return tuple(new)

        acc = lax.fori_loop(0, _PCH, _row, acc)
        pltpu.sync_copy(rows_v, hp_hbm.at[pl.ds(row0, _PCH)])
        return acc

    acc = lax.fori_loop(0, ROWS_W // _PCH, _chunk, acc0)

    for j in range(H // 16):
        obuf[pl.ds(j * 16, 16)] = acc[j]
    pltpu.sync_copy(obuf, pmax_hbm.at[c, s])
    for j in range(H // 16):
        obuf[pl.ds(j * 16, 16)] = acc[H // 16 + j]
    pltpu.sync_copy(obuf, psum_hbm.at[c, s])


@jax.jit
def _pool_pass(hn, g, sel):
    mesh = plsc.VectorSubcoreMesh(core_axis_name="c", subcore_axis_name="s")
    return pl.kernel(
        _pool_body,
        out_type=(
            jax.ShapeDtypeStruct((NP, H), jnp.float32),
            jax.ShapeDtypeStruct((NC, NS, H), jnp.float32),
            jax.ShapeDtypeStruct((NC, NS, H), jnp.float32),
        ),
        mesh=mesh,
        compiler_params=pltpu.CompilerParams(needs_layout_passes=False),
        scratch_types=[
            pltpu.VMEM((_PCH, H), jnp.float32),
            pltpu.VMEM((_PCH,), jnp.float32),
            pltpu.VMEM((_PCH,), jnp.float32),
            pltpu.VMEM((H,), jnp.float32),
        ],
    )(hn, g, sel)


def _mm1_body(h_ref, w_ref, degc_ref, degs_ref, xt_ref, dc_ref, ds_ref):
    xt_ref[...] = jnp.dot(h_ref[...], w_ref[...], preferred_element_type=jnp.float32)
    dc_ref[...] = lax.rsqrt(degc_ref[0] + degc_ref[1] + 1.0)
    ds_ref[...] = lax.rsqrt(degs_ref[0] + degs_ref[1] + 1.0)


@jax.jit
def _mm1_prep(h, W, degc2, degs2):
    return pl.pallas_call(
        _mm1_body,
        out_shape=(
            jax.ShapeDtypeStruct((NP, H), jnp.float32),
            jax.ShapeDtypeStruct((NR, H), jnp.float32),
            jax.ShapeDtypeStruct((NR, H), jnp.float32),
        ),
    )(h, W, degc2, degs2)


def _mm2_body(agg_ref, xt_ref, dc_ref, b_ref, wp_ref, hn_ref, st_ref):
    dc = dc_ref[...]
    hn = jax.nn.relu(dc * (agg_ref[0] + agg_ref[1]) + dc * dc * xt_ref[...]
                     + b_ref[...][None, :])
    hn_ref[...] = hn
    st_ref[...] = jnp.dot(hn, wp_ref[...], preferred_element_type=jnp.float32)


@jax.jit
def _mm2(agg, xt, dc_col, b, Wp):
    return pl.pallas_call(
        _mm2_body,
        out_shape=(
            jax.ShapeDtypeStruct((NP, H), jnp.float32),
            jax.ShapeDtypeStruct((NP, 1), jnp.float32),
        ),
    )(agg, xt, dc_col, b, Wp)


def _topk_body(k, sagg_ref, st_ref, dinvs_ref, m_ref, bp_ref, sel_ref, g_ref):
    dinvs = dinvs_ref[...]
    st = st_ref[...]
    score = dinvs * (sagg_ref[0] + sagg_ref[1]) + dinvs * dinvs * st + bp_ref[0, 0]
    scorem = jnp.where(m_ref[...] > 0, score, -jnp.inf)

    bits = lax.bitcast_convert_type(scorem, jnp.int32)
    key = jnp.where(bits >= 0, bits, bits ^ jnp.int32(0x7FFFFFFF))

    cnt_nonneg = jnp.sum((key >= 0).astype(jnp.int32))
    t0 = jnp.where(cnt_nonneg >= k, jnp.int32(0), jnp.int32(-2147483648))

    def _tstep(i, t):
        cand = t | lax.shift_left(jnp.int32(1), jnp.int32(30) - i)
        cnt = jnp.sum((key >= cand).astype(jnp.int32))
        return jnp.where(cnt >= k, cand, t)

    t = lax.fori_loop(0, 31, _tstep, t0)

    cnt_gt = jnp.sum((key > t).astype(jnp.int32))
    need = k - cnt_gt
    idx = (lax.broadcasted_iota(jnp.int32, (NR, H), 0) * H
           + lax.broadcasted_iota(jnp.int32, (NR, H), 1))
    eq = key == t

    def _jstep(i, jv):
        cand = jv | lax.shift_left(jnp.int32(1), jnp.int32(13) - i)
        cnt = jnp.sum((eq & (idx < cand)).astype(jnp.int32))
        return jnp.where(cnt <= need, cand, jv)

    j0 = lax.fori_loop(0, 14, _jstep, jnp.int32(0))

    sel = (key > t) | (eq & (idx < j0))
    self32 = sel.astype(jnp.float32)
    sel_ref[...] = self32
    g_ref[...] = jnp.tanh(score) * self32


import functools


@functools.partial(jax.jit, static_argnums=0)
def _topk_g(k, sagg2, st2, dinvs2, m2, bp):
    return pl.pallas_call(
        functools.partial(_topk_body, k),
        out_shape=(
            jax.ShapeDtypeStruct((NR, H), jnp.float32),
            jax.ShapeDtypeStruct((NR, H), jnp.float32),
        ),
    )(sagg2, st2, dinvs2, m2, bp)


def _head_body(k1, k2, k3, pmax_ref, psum_ref, wl1_ref, bl1_ref, wl2_ref,
               bl2_ref, wl3_ref, bl3_ref, o_ref):
    zs = []
    for l, kk in enumerate((k1, k2, k3)):
        mx = jnp.max(pmax_ref[l], axis=0, keepdims=True)
        mean = jnp.sum(psum_ref[l], axis=0, keepdims=True) / kk
        zs.append(jnp.concatenate([mx, mean], axis=1))
    z = zs[0] + zs[1] + zs[2]
    z = jax.nn.relu(z @ wl1_ref[...] + bl1_ref[...][None, :])
    z = jax.nn.relu(z @ wl2_ref[...] + bl2_ref[...][None, :])
    z = z @ wl3_ref[...] + bl3_ref[...][None, :]
    o_ref[...] = jax.nn.log_softmax(z, axis=-1)


@functools.partial(jax.jit, static_argnums=(0, 1, 2))
def _head(k1, k2, k3, pmax3, psum3, Wl1, bl1, Wl2, bl2, Wl3, bl3):
    return pl.pallas_call(
        functools.partial(_head_body, k1, k2, k3),
        out_shape=jax.ShapeDtypeStruct((1, 2), jnp.float32),
    )(pmax3, psum3, Wl1, bl1, Wl2, bl2, Wl3, bl3)


def _pad_nodes(a):
    return jnp.pad(a, ((0, NP - N),) + ((0, 0),) * (a.ndim - 1))


def kernel(x, edge_index, edge_attr, batch, W1, b1, Wp1, bp1, W2, b2, Wp2, bp2, W3, b3, Wp3, bp3, Wl1, bl1, Wl2, bl2, Wl3, bl3):
    src = edge_index[0].astype(jnp.int32)
    dst = edge_index[1].astype(jnp.int32)
    pad_n = EP - E
    pad_idx = N + (jnp.arange(pad_n, dtype=jnp.int32) % (NP - N))
    src_p = jnp.concatenate([src, pad_idx])
    dst_p = jnp.concatenate([dst, pad_idx])
    dst2dw = dst_p.reshape(EP // CH, CH)
    dst2de = dst_p.reshape(EP // ECH, ECH)
    attr_p = jnp.concatenate([edge_attr, jnp.zeros((pad_n,), jnp.float32)])
    ones_p = jnp.concatenate([jnp.ones((E,), jnp.float32), jnp.zeros((pad_n,), jnp.float32)])

    m = _pad_nodes(jnp.ones((N,), jnp.float32))
    h = _pad_nodes(x)

    ks, n = [], N
    for _ in range(3):
        ks.append(int(math.ceil(0.8 * n)))
        n = ks[-1]
    Ws = [(W1, b1, Wp1, bp1, attr_p), (W2, b2, Wp2, bp2, attr_p),
          (W3, b3, Wp3, bp3, ones_p)]

    pmaxs, psums = [], []
    for li, (W, b, Wp, bp, attr_l) in enumerate(Ws):
        k = ks[li]
        vn, ew, degc2, degs2, csrc, cdst, cew, cnts = _edge_pass(src_p, dst2de, attr_l, m)
        xt, dc2, ds2 = _mm1_prep(h, W, degc2.reshape(NC, NR, H), degs2.reshape(NC, NR, H))
        dinv_c = dc2.reshape(NP)
        dinv_s = ds2.reshape(NP)
        agg = _wide_pass(csrc, cdst.reshape(EP // CH, CH), cew, xt, dinv_c, cnts)
        hn, st = _mm2(agg, xt, jnp.broadcast_to(dinv_c.reshape(NP, 1), (NP, H)), b, Wp)
        st_flat = st.reshape(NP)
        sagg2 = _score_pass(src_p, dst2de, vn, st_flat, dinv_s)
        sel2, g2 = _topk_g(k, sagg2.reshape(NC, NR, H), st_flat.reshape(NR, H),
                           ds2, m.reshape(NR, H), bp.reshape(1, 1))
        sel = sel2.reshape(NP)
        h, pmax, psum = _pool_pass(hn, g2.reshape(NP), sel)
        pmaxs.append(pmax.reshape(NC * NS, H))
        psums.append(psum.reshape(NC * NS, H))
        m = sel

    return _head(float(ks[0]), float(ks[1]), float(ks[2]),
                 jnp.stack(pmaxs), jnp.stack(psums), Wl1, bl1, Wl2, bl2, Wl3, bl3)
